# baseline (speedup 1.0000x reference)
import functools

import jax
import jax.numpy as jnp
from jax import lax
from jax.experimental import pallas as pl
from jax.experimental.pallas import tpu as pltpu
from jax.experimental.pallas import tpu_sc as plsc

B, L, H = 4, 2048, 1024
N = B * L
NC, NS = 2, 16
NW = NC * NS
ROWS_PER_W = N // NW
C = 16
NCHUNK = ROWS_PER_W // C
LANES = 16


def kernel(X, W_word, W_pos):
    idx = X.reshape(N).astype(jnp.int32)
    mesh = plsc.VectorSubcoreMesh(core_axis_name="c", subcore_axis_name="s")

    @functools.partial(
        pl.kernel,
        out_type=jax.ShapeDtypeStruct((N, H), jnp.float32),
        mesh=mesh,
        scratch_types=[
            pltpu.VMEM((ROWS_PER_W,), jnp.int32),
            pltpu.VMEM((C, H), jnp.float32),
            pltpu.VMEM((C, H), jnp.float32),
            pltpu.VMEM((C, H), jnp.float32),
            pltpu.VMEM((C, H), jnp.float32),
            pltpu.VMEM((C, H), jnp.float32),
            pltpu.VMEM((C, H), jnp.float32),
            pltpu.SemaphoreType.DMA,
            pltpu.SemaphoreType.DMA,
            pltpu.SemaphoreType.DMA,
            pltpu.SemaphoreType.DMA,
            pltpu.SemaphoreType.DMA,
            pltpu.SemaphoreType.DMA,
        ],
    )
    def embed(w_hbm, p_hbm, i_hbm, o_hbm,
              idx_v, rows0, rows1, pos0, pos1, ob0, ob1,
              sg0, sg1, sp0, sp1, so0, so1):
        rows = [rows0, rows1]
        pos = [pos0, pos1]
        ob = [ob0, ob1]
        sg = [sg0, sg1]
        sp = [sp0, sp1]
        so = [so0, so1]

        wid = lax.axis_index("s") * NC + lax.axis_index("c")
        base = wid * ROWS_PER_W
        l0 = (wid % (L // ROWS_PER_W)) * ROWS_PER_W

        pltpu.sync_copy(i_hbm.at[pl.ds(base, ROWS_PER_W)], idx_v)

        def start(k):
            p = k % 2
            g = pltpu.async_copy(
                w_hbm.at[idx_v.at[pl.ds(k * C, C)]], rows[p], sg[p])
            q = pltpu.async_copy(
                p_hbm.at[pl.ds(l0 + k * C, C)], pos[p], sp[p])
            return g, q

        inflight = {0: start(0), 1: start(1)}
        out_cp = {}

        for k in range(NCHUNK):
            p = k % 2
            g, q = inflight.pop(k)
            g.wait()
            q.wait()
            if k >= 2:
                out_cp.pop(k - 2).wait()

            @pl.loop(0, C)
            def _(r):
                @pl.loop(0, H, step=LANES)
                def _(c):
                    s = pl.ds(c, LANES)
                    ob[p].at[r, s][...] = jnp.maximum(
                        rows[p].at[r, s][...] + pos[p].at[r, s][...], 0.0
                    )

            out_cp[k] = pltpu.async_copy(
                ob[p], o_hbm.at[pl.ds(base + k * C, C)], so[p])
            if k + 2 < NCHUNK:
                inflight[k + 2] = start(k + 2)

        out_cp.pop(NCHUNK - 2).wait()
        out_cp.pop(NCHUNK - 1).wait()

    out = embed(W_word, W_pos, idx)
    return out.reshape(B, L, H)
